# baseline (speedup 1.0000x reference)
import functools

import jax
import jax.numpy as jnp
from jax import lax
from jax.experimental import pallas as pl
from jax.experimental.pallas import tpu as pltpu
from jax.experimental.pallas import tpu_sc as plsc

B, F, D = 16384, 26, 16
FIELD_SIZE = 38462
D_IN = F * D

_NC, _NS = 2, 16
_NW = _NC * _NS
_BF = B * F
_IDX_W = 128
_ROWS_TOTAL = _BF // _IDX_W
_ROWS_PER_W = _ROWS_TOTAL // _NW
_CG = 8
_NCHUNK = _ROWS_PER_W // _CG


V = F * FIELD_SIZE
_NB_FULL = V // 128
_TAIL = V - _NB_FULL * 128


def _make_sc_transpose():
    mesh = plsc.VectorSubcoreMesh(
        core_axis_name="c", subcore_axis_name="s",
        num_cores=_NC, num_subcores=_NS)

    CW = 512
    NBLK = V // CW
    OW = CW * 16

    @functools.partial(
        pl.kernel,
        out_type=jax.ShapeDtypeStruct((V * D,), jnp.float32),
        mesh=mesh,
        scratch_types=(
            pltpu.VMEM((32, CW + 9), jnp.float32),
            pltpu.VMEM((2 * OW,), jnp.float32),
            pltpu.SemaphoreType.DMA,
            pltpu.SemaphoreType.DMA,
        ),
        compiler_params=pltpu.CompilerParams(use_tc_tiling_on_sc=True,
                                             needs_layout_passes=False,
                                             disable_bounds_checks=True),
    )
    def transpose_k(embT, tail_in, out, tbuf, obuf, sem_in, sem_out):
        wid = lax.axis_index("s") * _NC + lax.axis_index("c")
        nb = 61 + (wid < 1).astype(jnp.int32)
        base = wid * 61 + jnp.minimum(wid, 1)
        d16 = lax.broadcasted_iota(jnp.int32, (16,), 0)

        def issue_in(i, p):
            c0 = pl.multiple_of((base + i) * CW, CW)
            pltpu.async_copy(embT.at[pl.ds(0, 8), pl.ds(c0, CW)],
                             tbuf.at[pl.ds(p * 16, 8), pl.ds(0, CW)], sem_in)
            pltpu.async_copy(embT.at[pl.ds(8, 8), pl.ds(c0, CW)],
                             tbuf.at[pl.ds(p * 16 + 8, 8), pl.ds(0, CW)],
                             sem_in)

        def wait_in(p):
            for tr in range(2):
                pltpu.make_async_copy(
                    embT.at[pl.ds(0, 8), pl.ds(0, CW)],
                    tbuf.at[pl.ds(p * 16 + tr * 8, 8), pl.ds(0, CW)],
                    sem_in).wait()

        def wait_out():
            pltpu.make_async_copy(
                obuf.at[pl.ds(0, OW)], out.at[pl.ds(0, OW)],
                sem_out).wait()

        issue_in(0, 0)

        def blk(i, carry):
            p = i & 1
            wait_in(p)

            @pl.when(i + 1 < nb)
            def _():
                issue_in(i + 1, p ^ 1)

            @pl.when(i >= 2)
            def _():
                wait_out()

            row0 = jnp.full((16,), p * 16, jnp.int32) + d16
            obase = p * OW

            @plsc.parallel_loop(0, CW, 1, unroll=16)
            def _cols(c):
                cv = jnp.full((16,), 0, jnp.int32) + c
                v = plsc.load_gather(tbuf, [row0, cv])
                obuf[pl.ds(obase + c * 16, 16)] = v
            c0 = pl.multiple_of((base + i) * CW, CW)
            pltpu.async_copy(obuf.at[pl.ds(p * OW, OW)],
                             out.at[pl.ds(c0 * 16, OW)], sem_out)
            return carry

        lax.fori_loop(0, nb, blk, 0)
        wait_out()

        @pl.when(nb >= 2)
        def _():
            wait_out()

        @pl.when(wid == 31)
        def _():
            pltpu.sync_copy(tail_in, obuf.at[pl.ds(0, _TAIL * 16)])
            pltpu.sync_copy(obuf.at[pl.ds(0, _TAIL * 16)],
                            out.at[pl.ds(_NB_FULL * 2048, _TAIL * 16)])

    return transpose_k


def _make_sc_gather():
    mesh = plsc.VectorSubcoreMesh(
        core_axis_name="c", subcore_axis_name="s",
        num_cores=_NC, num_subcores=_NS)

    @functools.partial(
        pl.kernel,
        out_type=(
            jax.ShapeDtypeStruct((_BF, D), jnp.float32),
            jax.ShapeDtypeStruct((_BF,), jnp.float32),
        ),
        mesh=mesh,
        scratch_types=(
            pltpu.VMEM((_ROWS_PER_W, _IDX_W), jnp.int32),
            pltpu.VMEM((_CG * _IDX_W, D), jnp.float32),
            pltpu.VMEM((_CG * _IDX_W,), jnp.float32),
            pltpu.SemaphoreType.DMA,
            pltpu.SemaphoreType.DMA,
        ),
        compiler_params=pltpu.CompilerParams(use_tc_tiling_on_sc=False),
    )
    def sc_gather(idx_hbm, emb_hbm, lin_hbm, rows_out, lin_out,
                  idx_v, rows_v, lin_v, sem_e, sem_l):
        wid = lax.axis_index("s") * _NC + lax.axis_index("c")
        row0 = wid * _ROWS_PER_W
        pltpu.sync_copy(idx_hbm.at[pl.ds(row0, _ROWS_PER_W)], idx_v)

        def chunk(g, carry):
            cps = []
            for b in range(_CG):
                r = g * _CG + b
                cps.append(pltpu.async_copy(
                    emb_hbm.at[idx_v.at[r]],
                    rows_v.at[pl.ds(b * _IDX_W, _IDX_W)], sem_e))
                cps.append(pltpu.async_copy(
                    lin_hbm.at[idx_v.at[r]],
                    lin_v.at[pl.ds(b * _IDX_W, _IDX_W)], sem_l))
            for cp in cps:
                cp.wait()
            base = (row0 + g * _CG) * _IDX_W
            pltpu.sync_copy(rows_v, rows_out.at[pl.ds(base, _CG * _IDX_W)])
            pltpu.sync_copy(lin_v, lin_out.at[pl.ds(base, _CG * _IDX_W)])
            return carry

        lax.fori_loop(0, _NCHUNK, chunk, 0)

    return sc_gather


_sc_cache = {}


def _sc_gather(idx2d, emb_table, lin_w):
    if "g" not in _sc_cache:
        _sc_cache["g"] = _make_sc_gather()
    return _sc_cache["g"](idx2d, emb_table, lin_w)


def _sc_transpose(embT, tail_in):
    if "t" not in _sc_cache:
        _sc_cache["t"] = _make_sc_transpose()
    return _sc_cache["t"](embT, tail_in)

_BB = 1024


def _tc_body(emb_ref, xv_ref, linv_ref, w1_ref, b1_ref, w2_ref, b2_ref,
             w3_ref, b3_ref, lb_ref, out_ref):
    ex = emb_ref[...]
    xv = xv_ref[...]
    f_ids = lax.broadcasted_iota(jnp.int32, (F, D_IN), 0)
    j_ids = lax.broadcasted_iota(jnp.int32, (F, D_IN), 1)
    e_mat = (lax.shift_right_logical(j_ids, 4) == f_ids).astype(jnp.float32)
    xve = jnp.dot(xv, e_mat, preferred_element_type=jnp.float32)
    ex = ex * xve
    j2 = lax.broadcasted_iota(jnp.int32, (D_IN, D), 0)
    d2 = lax.broadcasted_iota(jnp.int32, (D_IN, D), 1)
    s_mat = ((j2 & (D - 1)) == d2).astype(jnp.float32)
    s = jnp.dot(ex, s_mat, preferred_element_type=jnp.float32)
    sq = jnp.dot(ex * ex, s_mat, preferred_element_type=jnp.float32)
    fm = 0.5 * jnp.sum(s * s - sq, axis=1, keepdims=True)
    linear = jnp.sum(linv_ref[...] * xv, axis=1, keepdims=True) + lb_ref[0, 0]
    h = jnp.dot(ex, w1_ref[...], preferred_element_type=jnp.float32)
    h = jnp.maximum(h + b1_ref[...], 0.0)
    h = jnp.dot(h, w2_ref[...], preferred_element_type=jnp.float32)
    h = jnp.maximum(h + b2_ref[...], 0.0)
    mlp = jnp.dot(h, w3_ref[...], preferred_element_type=jnp.float32)
    out_ref[...] = linear + fm + mlp + b3_ref[0, 0]


def _tc_call(emb2d, xv, linv2d, w1, b1, w2, b2, w3, b3, lb):
    return pl.pallas_call(
        _tc_body,
        grid=(B // _BB,),
        in_specs=[
            pl.BlockSpec((_BB, D_IN), lambda i: (i, 0)),
            pl.BlockSpec((_BB, F), lambda i: (i, 0)),
            pl.BlockSpec((_BB, F), lambda i: (i, 0)),
            pl.BlockSpec((D_IN, 256), lambda i: (0, 0)),
            pl.BlockSpec((1, 256), lambda i: (0, 0)),
            pl.BlockSpec((256, 128), lambda i: (0, 0)),
            pl.BlockSpec((1, 128), lambda i: (0, 0)),
            pl.BlockSpec((128, 1), lambda i: (0, 0)),
            pl.BlockSpec((1, 1), lambda i: (0, 0)),
            pl.BlockSpec((1, 1), lambda i: (0, 0)),
        ],
        out_specs=pl.BlockSpec((_BB, 1), lambda i: (i, 0)),
        out_shape=jax.ShapeDtypeStruct((B, 1), jnp.float32),
        compiler_params=pltpu.CompilerParams(
            dimension_semantics=("parallel",)),
    )(emb2d, xv, linv2d, w1, b1, w2, b2, w3, b3, lb)


def kernel(x_field, x, x_val, emb_table, lin_w, lin_b, W1, b1, W2, b2, W3, b3):
    idx = x + x_field * FIELD_SIZE
    idx2d = idx.reshape(_ROWS_TOTAL, _IDX_W)
    tail = lax.slice(emb_table, (_NB_FULL * 128, 0), (V, D)).reshape(_TAIL * D)
    emb_lin = _sc_transpose(emb_table.T, tail).reshape(V, D)
    rows, linv = _sc_gather(idx2d, emb_lin, lin_w[:, 0])
    emb2d = rows.reshape(B, D_IN)
    linv2d = linv.reshape(B, F)
    out = _tc_call(
        emb2d, x_val, linv2d, W1, b1.reshape(1, 256), W2, b2.reshape(1, 128),
        W3, b3.reshape(1, 1), lin_b.reshape(1, 1))
    return out[:, 0]


# speedup vs baseline: 1.0560x; 1.0560x over previous
import functools

import jax
import jax.numpy as jnp
from jax import lax
from jax.experimental import pallas as pl
from jax.experimental.pallas import tpu as pltpu
from jax.experimental.pallas import tpu_sc as plsc

B, F, D = 16384, 26, 16
FIELD_SIZE = 38462
D_IN = F * D

_NC, _NS = 2, 16
_NW = _NC * _NS
_BF = B * F
_IDX_W = 128
_ROWS_TOTAL = _BF // _IDX_W
_ROWS_PER_W = _ROWS_TOTAL // _NW
_CG = 8
_NCHUNK = _ROWS_PER_W // _CG


V = F * FIELD_SIZE
_NB_FULL = V // 128
_TAIL = V - _NB_FULL * 128


def _make_sc_transpose():
    mesh = plsc.VectorSubcoreMesh(
        core_axis_name="c", subcore_axis_name="s",
        num_cores=_NC, num_subcores=_NS)

    CW = 128
    NBLK = V // CW
    NQ, NR = divmod(NBLK, _NW)
    OW = CW * 16

    @functools.partial(
        pl.kernel,
        out_type=jax.ShapeDtypeStruct((V * D,), jnp.float32),
        mesh=mesh,
        scratch_types=(
            pltpu.VMEM((32, CW + 9), jnp.float32),
            pltpu.VMEM((2 * OW,), jnp.float32),
            pltpu.SemaphoreType.DMA,
            pltpu.SemaphoreType.DMA,
        ),
        compiler_params=pltpu.CompilerParams(use_tc_tiling_on_sc=True,
                                             needs_layout_passes=False,
                                             disable_bounds_checks=True),
    )
    def transpose_k(embT, tail_in, out, tbuf, obuf, sem_in, sem_out):
        wid = lax.axis_index("s") * _NC + lax.axis_index("c")
        nb = NQ + (wid < NR).astype(jnp.int32)
        base = wid * NQ + jnp.minimum(wid, NR)
        d16 = lax.broadcasted_iota(jnp.int32, (16,), 0)

        def issue_in(i, p):
            c0 = pl.multiple_of((base + i) * CW, CW)
            pltpu.async_copy(embT.at[pl.ds(0, 8), pl.ds(c0, CW)],
                             tbuf.at[pl.ds(p * 16, 8), pl.ds(0, CW)], sem_in)
            pltpu.async_copy(embT.at[pl.ds(8, 8), pl.ds(c0, CW)],
                             tbuf.at[pl.ds(p * 16 + 8, 8), pl.ds(0, CW)],
                             sem_in)

        def wait_in(p):
            for tr in range(2):
                pltpu.make_async_copy(
                    embT.at[pl.ds(0, 8), pl.ds(0, CW)],
                    tbuf.at[pl.ds(p * 16 + tr * 8, 8), pl.ds(0, CW)],
                    sem_in).wait()

        def wait_out():
            pltpu.make_async_copy(
                obuf.at[pl.ds(0, OW)], out.at[pl.ds(0, OW)],
                sem_out).wait()

        issue_in(0, 0)

        def blk(i, carry):
            p = i & 1
            wait_in(p)

            @pl.when(i + 1 < nb)
            def _():
                issue_in(i + 1, p ^ 1)

            @pl.when(i >= 2)
            def _():
                wait_out()

            row0 = jnp.full((16,), p * 16, jnp.int32) + d16
            obase = p * OW

            @plsc.parallel_loop(0, CW, 1, unroll=16)
            def _cols(c):
                cv = jnp.full((16,), 0, jnp.int32) + c
                v = plsc.load_gather(tbuf, [row0, cv])
                obuf[pl.ds(obase + c * 16, 16)] = v
            c0 = pl.multiple_of((base + i) * CW, CW)
            pltpu.async_copy(obuf.at[pl.ds(p * OW, OW)],
                             out.at[pl.ds(c0 * 16, OW)], sem_out)
            return carry

        lax.fori_loop(0, nb, blk, 0)
        wait_out()

        @pl.when(nb >= 2)
        def _():
            wait_out()

        @pl.when(wid == 31)
        def _():
            pltpu.sync_copy(tail_in, obuf.at[pl.ds(0, _TAIL * 16)])
            pltpu.sync_copy(obuf.at[pl.ds(0, _TAIL * 16)],
                            out.at[pl.ds(_NB_FULL * 2048, _TAIL * 16)])

    return transpose_k


def _make_sc_gather():
    mesh = plsc.VectorSubcoreMesh(
        core_axis_name="c", subcore_axis_name="s",
        num_cores=_NC, num_subcores=_NS)

    @functools.partial(
        pl.kernel,
        out_type=(
            jax.ShapeDtypeStruct((_BF, D), jnp.float32),
            jax.ShapeDtypeStruct((_BF,), jnp.float32),
        ),
        mesh=mesh,
        scratch_types=(
            pltpu.VMEM((_ROWS_PER_W, _IDX_W), jnp.int32),
            pltpu.VMEM((2 * _CG * _IDX_W, D), jnp.float32),
            pltpu.VMEM((2 * _CG * _IDX_W,), jnp.float32),
            pltpu.SemaphoreType.DMA,
            pltpu.SemaphoreType.DMA,
            pltpu.SemaphoreType.DMA,
        ),
        compiler_params=pltpu.CompilerParams(use_tc_tiling_on_sc=False),
    )
    def sc_gather(idx_hbm, emb_hbm, lin_hbm, rows_out, lin_out,
                  idx_v, rows_v, lin_v, sem_e, sem_l, sem_o):
        wid = lax.axis_index("s") * _NC + lax.axis_index("c")
        row0 = wid * _ROWS_PER_W
        CHW = _CG * _IDX_W
        pltpu.sync_copy(idx_hbm.at[pl.ds(row0, _ROWS_PER_W)], idx_v)

        def issue(g, p):
            for b in range(_CG):
                r = g * _CG + b
                pltpu.async_copy(
                    emb_hbm.at[idx_v.at[r]],
                    rows_v.at[pl.ds(p * CHW + b * _IDX_W, _IDX_W)], sem_e)
                pltpu.async_copy(
                    lin_hbm.at[idx_v.at[r]],
                    lin_v.at[pl.ds(p * CHW + b * _IDX_W, _IDX_W)], sem_l)

        def wait_in(p):
            for b in range(_CG):
                pltpu.make_async_copy(
                    emb_hbm.at[idx_v.at[0]],
                    rows_v.at[pl.ds(p * CHW + b * _IDX_W, _IDX_W)],
                    sem_e).wait()
                pltpu.make_async_copy(
                    lin_hbm.at[idx_v.at[0]],
                    lin_v.at[pl.ds(p * CHW + b * _IDX_W, _IDX_W)],
                    sem_l).wait()

        def wait_out():
            pltpu.make_async_copy(
                rows_v.at[pl.ds(0, CHW)], rows_out.at[pl.ds(0, CHW)],
                sem_o).wait()
            pltpu.make_async_copy(
                lin_v.at[pl.ds(0, CHW)], lin_out.at[pl.ds(0, CHW)],
                sem_o).wait()

        issue(0, 0)

        def chunk(g, carry):
            p = g & 1
            wait_in(p)

            @pl.when(g + 1 < _NCHUNK)
            def _():
                issue(g + 1, p ^ 1)

            @pl.when(g >= 2)
            def _():
                wait_out()

            base = (row0 + g * _CG) * _IDX_W
            pltpu.async_copy(rows_v.at[pl.ds(p * CHW, CHW)],
                             rows_out.at[pl.ds(base, CHW)], sem_o)
            pltpu.async_copy(lin_v.at[pl.ds(p * CHW, CHW)],
                             lin_out.at[pl.ds(base, CHW)], sem_o)
            return carry

        lax.fori_loop(0, _NCHUNK, chunk, 0)
        wait_out()
        wait_out()

    return sc_gather


_sc_cache = {}


def _sc_gather(idx2d, emb_table, lin_w):
    if "g" not in _sc_cache:
        _sc_cache["g"] = _make_sc_gather()
    return _sc_cache["g"](idx2d, emb_table, lin_w)


def _sc_transpose(embT, tail_in):
    if "t" not in _sc_cache:
        _sc_cache["t"] = _make_sc_transpose()
    return _sc_cache["t"](embT, tail_in)

_BB = 1024


def _tc_body(emb_ref, xv_ref, linv_ref, w1_ref, b1_ref, w2_ref, b2_ref,
             w3_ref, b3_ref, lb_ref, out_ref):
    ex = emb_ref[...]
    xv = xv_ref[...]
    f_ids = lax.broadcasted_iota(jnp.int32, (F, D_IN), 0)
    j_ids = lax.broadcasted_iota(jnp.int32, (F, D_IN), 1)
    e_mat = (lax.shift_right_logical(j_ids, 4) == f_ids).astype(jnp.float32)
    xve = jnp.dot(xv, e_mat, preferred_element_type=jnp.float32)
    ex = ex * xve
    j2 = lax.broadcasted_iota(jnp.int32, (D_IN, D), 0)
    d2 = lax.broadcasted_iota(jnp.int32, (D_IN, D), 1)
    s_mat = ((j2 & (D - 1)) == d2).astype(jnp.float32)
    s = jnp.dot(ex, s_mat, preferred_element_type=jnp.float32)
    sq = jnp.dot(ex * ex, s_mat, preferred_element_type=jnp.float32)
    fm = 0.5 * jnp.sum(s * s - sq, axis=1, keepdims=True)
    linear = jnp.sum(linv_ref[...] * xv, axis=1, keepdims=True) + lb_ref[0, 0]
    h = jnp.dot(ex, w1_ref[...], preferred_element_type=jnp.float32)
    h = jnp.maximum(h + b1_ref[...], 0.0)
    h = jnp.dot(h, w2_ref[...], preferred_element_type=jnp.float32)
    h = jnp.maximum(h + b2_ref[...], 0.0)
    mlp = jnp.dot(h, w3_ref[...], preferred_element_type=jnp.float32)
    out_ref[...] = linear + fm + mlp + b3_ref[0, 0]


def _tc_call(emb2d, xv, linv2d, w1, b1, w2, b2, w3, b3, lb):
    return pl.pallas_call(
        _tc_body,
        grid=(B // _BB,),
        in_specs=[
            pl.BlockSpec((_BB, D_IN), lambda i: (i, 0)),
            pl.BlockSpec((_BB, F), lambda i: (i, 0)),
            pl.BlockSpec((_BB, F), lambda i: (i, 0)),
            pl.BlockSpec((D_IN, 256), lambda i: (0, 0)),
            pl.BlockSpec((1, 256), lambda i: (0, 0)),
            pl.BlockSpec((256, 128), lambda i: (0, 0)),
            pl.BlockSpec((1, 128), lambda i: (0, 0)),
            pl.BlockSpec((128, 1), lambda i: (0, 0)),
            pl.BlockSpec((1, 1), lambda i: (0, 0)),
            pl.BlockSpec((1, 1), lambda i: (0, 0)),
        ],
        out_specs=pl.BlockSpec((_BB, 1), lambda i: (i, 0)),
        out_shape=jax.ShapeDtypeStruct((B, 1), jnp.float32),
        compiler_params=pltpu.CompilerParams(
            dimension_semantics=("parallel",)),
    )(emb2d, xv, linv2d, w1, b1, w2, b2, w3, b3, lb)


def kernel(x_field, x, x_val, emb_table, lin_w, lin_b, W1, b1, W2, b2, W3, b3):
    idx = x + x_field * FIELD_SIZE
    idx2d = idx.reshape(_ROWS_TOTAL, _IDX_W)
    tail = lax.slice(emb_table, (_NB_FULL * 128, 0), (V, D)).reshape(_TAIL * D)
    emb_lin = _sc_transpose(emb_table.T, tail).reshape(V, D)
    rows, linv = _sc_gather(idx2d, emb_lin, lin_w[:, 0])
    emb2d = rows.reshape(B, D_IN)
    linv2d = linv.reshape(B, F)
    out = _tc_call(
        emb2d, x_val, linv2d, W1, b1.reshape(1, 256), W2, b2.reshape(1, 128),
        W3, b3.reshape(1, 1), lin_b.reshape(1, 1))
    return out[:, 0]


# speedup vs baseline: 1.0738x; 1.0168x over previous
import functools

import jax
import jax.numpy as jnp
from jax import lax
from jax.experimental import pallas as pl
from jax.experimental.pallas import tpu as pltpu
from jax.experimental.pallas import tpu_sc as plsc

B, F, D = 16384, 26, 16
FIELD_SIZE = 38462
D_IN = F * D

_NC, _NS = 2, 16
_NW = _NC * _NS
_BF = B * F
_IDX_W = 128
_ROWS_TOTAL = _BF // _IDX_W
_ROWS_PER_W = _ROWS_TOTAL // _NW
_CG = 8
_NCHUNK = _ROWS_PER_W // _CG


V = F * FIELD_SIZE
_NB_FULL = V // 128
_TAIL = V - _NB_FULL * 128


def _make_sc_transpose():
    mesh = plsc.VectorSubcoreMesh(
        core_axis_name="c", subcore_axis_name="s",
        num_cores=_NC, num_subcores=_NS)

    CW = 128
    NBLK = V // CW
    NQ, NR = divmod(NBLK, _NW)
    OW = CW * 16

    @functools.partial(
        pl.kernel,
        out_type=jax.ShapeDtypeStruct((V * D,), jnp.float32),
        mesh=mesh,
        scratch_types=(
            pltpu.VMEM((64, CW + 9), jnp.float32),
            pltpu.VMEM((4 * OW,), jnp.float32),
            pltpu.SemaphoreType.DMA,
            pltpu.SemaphoreType.DMA,
        ),
        compiler_params=pltpu.CompilerParams(use_tc_tiling_on_sc=True,
                                             needs_layout_passes=False,
                                             disable_bounds_checks=True),
    )
    def transpose_k(embT, tail_in, out, tbuf, obuf, sem_in, sem_out):
        wid = lax.axis_index("s") * _NC + lax.axis_index("c")
        nb = NQ + (wid < NR).astype(jnp.int32)
        base = wid * NQ + jnp.minimum(wid, NR)
        d16 = lax.broadcasted_iota(jnp.int32, (16,), 0)

        def issue_in(i, p):
            c0 = pl.multiple_of((base + i) * CW, CW)
            pltpu.async_copy(embT.at[pl.ds(0, 8), pl.ds(c0, CW)],
                             tbuf.at[pl.ds(p * 16, 8), pl.ds(0, CW)], sem_in)
            pltpu.async_copy(embT.at[pl.ds(8, 8), pl.ds(c0, CW)],
                             tbuf.at[pl.ds(p * 16 + 8, 8), pl.ds(0, CW)],
                             sem_in)

        def wait_in(p):
            for tr in range(2):
                pltpu.make_async_copy(
                    embT.at[pl.ds(0, 8), pl.ds(0, CW)],
                    tbuf.at[pl.ds(p * 16 + tr * 8, 8), pl.ds(0, CW)],
                    sem_in).wait()

        def wait_out():
            pltpu.make_async_copy(
                obuf.at[pl.ds(0, OW)], out.at[pl.ds(0, OW)],
                sem_out).wait()

        issue_in(0, 0)
        issue_in(1, 1)
        issue_in(2, 2)

        def blk(i, carry):
            p = i & 3
            wait_in(p)

            @pl.when(i + 3 < nb)
            def _():
                issue_in(i + 3, (i + 3) & 3)

            @pl.when(i >= 4)
            def _():
                wait_out()

            row0 = jnp.full((16,), p * 16, jnp.int32) + d16
            obase = p * OW

            @plsc.parallel_loop(0, CW, 1, unroll=16)
            def _cols(c):
                cv = jnp.full((16,), 0, jnp.int32) + c
                v = plsc.load_gather(tbuf, [row0, cv])
                obuf[pl.ds(obase + c * 16, 16)] = v
            c0 = pl.multiple_of((base + i) * CW, CW)
            pltpu.async_copy(obuf.at[pl.ds(p * OW, OW)],
                             out.at[pl.ds(c0 * 16, OW)], sem_out)
            return carry

        lax.fori_loop(0, nb, blk, 0)
        for _i in range(4):
            wait_out()

        @pl.when(wid == 31)
        def _():
            pltpu.sync_copy(tail_in, obuf.at[pl.ds(0, _TAIL * 16)])
            pltpu.sync_copy(obuf.at[pl.ds(0, _TAIL * 16)],
                            out.at[pl.ds(_NB_FULL * 2048, _TAIL * 16)])

    return transpose_k


def _make_sc_gather():
    mesh = plsc.VectorSubcoreMesh(
        core_axis_name="c", subcore_axis_name="s",
        num_cores=_NC, num_subcores=_NS)

    @functools.partial(
        pl.kernel,
        out_type=(
            jax.ShapeDtypeStruct((_BF, D), jnp.float32),
            jax.ShapeDtypeStruct((_BF,), jnp.float32),
        ),
        mesh=mesh,
        scratch_types=(
            pltpu.VMEM((_ROWS_PER_W, _IDX_W), jnp.int32),
            pltpu.VMEM((2 * _CG * _IDX_W, D), jnp.float32),
            pltpu.VMEM((2 * _CG * _IDX_W,), jnp.float32),
            pltpu.SemaphoreType.DMA,
            pltpu.SemaphoreType.DMA,
            pltpu.SemaphoreType.DMA,
        ),
        compiler_params=pltpu.CompilerParams(use_tc_tiling_on_sc=False),
    )
    def sc_gather(idx_hbm, emb_hbm, lin_hbm, rows_out, lin_out,
                  idx_v, rows_v, lin_v, sem_e, sem_l, sem_o):
        wid = lax.axis_index("s") * _NC + lax.axis_index("c")
        row0 = wid * _ROWS_PER_W
        CHW = _CG * _IDX_W
        pltpu.sync_copy(idx_hbm.at[pl.ds(row0, _ROWS_PER_W)], idx_v)

        def issue(g, p):
            for b in range(_CG):
                r = g * _CG + b
                pltpu.async_copy(
                    emb_hbm.at[idx_v.at[r]],
                    rows_v.at[pl.ds(p * CHW + b * _IDX_W, _IDX_W)], sem_e)
                pltpu.async_copy(
                    lin_hbm.at[idx_v.at[r]],
                    lin_v.at[pl.ds(p * CHW + b * _IDX_W, _IDX_W)], sem_l)

        def wait_in(p):
            for b in range(_CG):
                pltpu.make_async_copy(
                    emb_hbm.at[idx_v.at[0]],
                    rows_v.at[pl.ds(p * CHW + b * _IDX_W, _IDX_W)],
                    sem_e).wait()
                pltpu.make_async_copy(
                    lin_hbm.at[idx_v.at[0]],
                    lin_v.at[pl.ds(p * CHW + b * _IDX_W, _IDX_W)],
                    sem_l).wait()

        def wait_out():
            pltpu.make_async_copy(
                rows_v.at[pl.ds(0, CHW)], rows_out.at[pl.ds(0, CHW)],
                sem_o).wait()
            pltpu.make_async_copy(
                lin_v.at[pl.ds(0, CHW)], lin_out.at[pl.ds(0, CHW)],
                sem_o).wait()

        issue(0, 0)

        def chunk(g, carry):
            p = g & 1
            wait_in(p)

            @pl.when(g + 1 < _NCHUNK)
            def _():
                issue(g + 1, p ^ 1)

            @pl.when(g >= 2)
            def _():
                wait_out()

            base = (row0 + g * _CG) * _IDX_W
            pltpu.async_copy(rows_v.at[pl.ds(p * CHW, CHW)],
                             rows_out.at[pl.ds(base, CHW)], sem_o)
            pltpu.async_copy(lin_v.at[pl.ds(p * CHW, CHW)],
                             lin_out.at[pl.ds(base, CHW)], sem_o)
            return carry

        lax.fori_loop(0, _NCHUNK, chunk, 0)
        wait_out()
        wait_out()

    return sc_gather


_sc_cache = {}


def _sc_gather(idx2d, emb_table, lin_w):
    if "g" not in _sc_cache:
        _sc_cache["g"] = _make_sc_gather()
    return _sc_cache["g"](idx2d, emb_table, lin_w)


def _sc_transpose(embT, tail_in):
    if "t" not in _sc_cache:
        _sc_cache["t"] = _make_sc_transpose()
    return _sc_cache["t"](embT, tail_in)

_BB = 1024


def _tc_body(emb_ref, xv_ref, linv_ref, w1_ref, b1_ref, w2_ref, b2_ref,
             w3_ref, b3_ref, lb_ref, out_ref):
    ex = emb_ref[...]
    xv = xv_ref[...]
    f_ids = lax.broadcasted_iota(jnp.int32, (F, D_IN), 0)
    j_ids = lax.broadcasted_iota(jnp.int32, (F, D_IN), 1)
    e_mat = (lax.shift_right_logical(j_ids, 4) == f_ids).astype(jnp.float32)
    xve = jnp.dot(xv, e_mat, preferred_element_type=jnp.float32)
    ex = ex * xve
    j2 = lax.broadcasted_iota(jnp.int32, (D_IN, D), 0)
    d2 = lax.broadcasted_iota(jnp.int32, (D_IN, D), 1)
    s_mat = ((j2 & (D - 1)) == d2).astype(jnp.float32)
    s = jnp.dot(ex, s_mat, preferred_element_type=jnp.float32)
    sq = jnp.dot(ex * ex, s_mat, preferred_element_type=jnp.float32)
    fm = 0.5 * jnp.sum(s * s - sq, axis=1, keepdims=True)
    linear = jnp.sum(linv_ref[...] * xv, axis=1, keepdims=True) + lb_ref[0, 0]
    h = jnp.dot(ex, w1_ref[...], preferred_element_type=jnp.float32)
    h = jnp.maximum(h + b1_ref[...], 0.0)
    h = jnp.dot(h, w2_ref[...], preferred_element_type=jnp.float32)
    h = jnp.maximum(h + b2_ref[...], 0.0)
    mlp = jnp.dot(h, w3_ref[...], preferred_element_type=jnp.float32)
    out_ref[...] = linear + fm + mlp + b3_ref[0, 0]


def _tc_call(emb2d, xv, linv2d, w1, b1, w2, b2, w3, b3, lb):
    return pl.pallas_call(
        _tc_body,
        grid=(B // _BB,),
        in_specs=[
            pl.BlockSpec((_BB, D_IN), lambda i: (i, 0)),
            pl.BlockSpec((_BB, F), lambda i: (i, 0)),
            pl.BlockSpec((_BB, F), lambda i: (i, 0)),
            pl.BlockSpec((D_IN, 256), lambda i: (0, 0)),
            pl.BlockSpec((1, 256), lambda i: (0, 0)),
            pl.BlockSpec((256, 128), lambda i: (0, 0)),
            pl.BlockSpec((1, 128), lambda i: (0, 0)),
            pl.BlockSpec((128, 1), lambda i: (0, 0)),
            pl.BlockSpec((1, 1), lambda i: (0, 0)),
            pl.BlockSpec((1, 1), lambda i: (0, 0)),
        ],
        out_specs=pl.BlockSpec((_BB, 1), lambda i: (i, 0)),
        out_shape=jax.ShapeDtypeStruct((B, 1), jnp.float32),
        compiler_params=pltpu.CompilerParams(
            dimension_semantics=("parallel",)),
    )(emb2d, xv, linv2d, w1, b1, w2, b2, w3, b3, lb)


def kernel(x_field, x, x_val, emb_table, lin_w, lin_b, W1, b1, W2, b2, W3, b3):
    idx = x + x_field * FIELD_SIZE
    idx2d = idx.reshape(_ROWS_TOTAL, _IDX_W)
    tail = lax.slice(emb_table, (_NB_FULL * 128, 0), (V, D)).reshape(_TAIL * D)
    emb_lin = _sc_transpose(emb_table.T, tail).reshape(V, D)
    rows, linv = _sc_gather(idx2d, emb_lin, lin_w[:, 0])
    emb2d = rows.reshape(B, D_IN)
    linv2d = linv.reshape(B, F)
    out = _tc_call(
        emb2d, x_val, linv2d, W1, b1.reshape(1, 256), W2, b2.reshape(1, 128),
        W3, b3.reshape(1, 1), lin_b.reshape(1, 1))
    return out[:, 0]


# speedup vs baseline: 1.0858x; 1.0112x over previous
import functools

import jax
import jax.numpy as jnp
from jax import lax
from jax.experimental import pallas as pl
from jax.experimental.pallas import tpu as pltpu
from jax.experimental.pallas import tpu_sc as plsc

B, F, D = 16384, 26, 16
FIELD_SIZE = 38462
D_IN = F * D

_NC, _NS = 2, 16
_NW = _NC * _NS
_BF = B * F
_IDX_W = 128
_ROWS_TOTAL = _BF // _IDX_W
_ROWS_PER_W = _ROWS_TOTAL // _NW
_CG = 8
_NCHUNK = _ROWS_PER_W // _CG


V = F * FIELD_SIZE
_NB_FULL = V // 128
_TAIL = V - _NB_FULL * 128


def _make_sc_transpose():
    mesh = plsc.VectorSubcoreMesh(
        core_axis_name="c", subcore_axis_name="s",
        num_cores=_NC, num_subcores=_NS)

    CW = 128
    NBLK = V // CW
    NQ, NR = divmod(NBLK, _NW)
    OW = CW * 16

    @functools.partial(
        pl.kernel,
        out_type=jax.ShapeDtypeStruct((V * D,), jnp.float32),
        mesh=mesh,
        scratch_types=(
            pltpu.VMEM((64, CW + 9), jnp.float32),
            pltpu.VMEM((4 * OW,), jnp.float32),
            pltpu.SemaphoreType.DMA,
            pltpu.SemaphoreType.DMA,
        ),
        compiler_params=pltpu.CompilerParams(use_tc_tiling_on_sc=True,
                                             needs_layout_passes=False,
                                             disable_bounds_checks=True),
    )
    def transpose_k(embT, tail_in, out, tbuf, obuf, sem_in, sem_out):
        wid = lax.axis_index("s") * _NC + lax.axis_index("c")
        nb = NQ + (wid < NR).astype(jnp.int32)
        base = wid * NQ + jnp.minimum(wid, NR)
        d16 = lax.broadcasted_iota(jnp.int32, (16,), 0)

        def issue_in(i, p):
            c0 = pl.multiple_of((base + i) * CW, CW)
            pltpu.async_copy(embT.at[pl.ds(0, 8), pl.ds(c0, CW)],
                             tbuf.at[pl.ds(p * 16, 8), pl.ds(0, CW)], sem_in)
            pltpu.async_copy(embT.at[pl.ds(8, 8), pl.ds(c0, CW)],
                             tbuf.at[pl.ds(p * 16 + 8, 8), pl.ds(0, CW)],
                             sem_in)

        def wait_in(p):
            for tr in range(2):
                pltpu.make_async_copy(
                    embT.at[pl.ds(0, 8), pl.ds(0, CW)],
                    tbuf.at[pl.ds(p * 16 + tr * 8, 8), pl.ds(0, CW)],
                    sem_in).wait()

        def wait_out():
            pltpu.make_async_copy(
                obuf.at[pl.ds(0, OW)], out.at[pl.ds(0, OW)],
                sem_out).wait()

        issue_in(0, 0)
        issue_in(1, 1)
        issue_in(2, 2)

        def blk(i, carry):
            p = i & 3
            wait_in(p)

            @pl.when(i + 3 < nb)
            def _():
                issue_in(i + 3, (i + 3) & 3)

            @pl.when(i >= 4)
            def _():
                wait_out()

            row0 = jnp.full((16,), p * 16, jnp.int32) + d16
            obase = p * OW

            @plsc.parallel_loop(0, CW, 1, unroll=16)
            def _cols(c):
                cv = jnp.full((16,), 0, jnp.int32) + c
                v = plsc.load_gather(tbuf, [row0, cv])
                obuf[pl.ds(obase + c * 16, 16)] = v
            c0 = pl.multiple_of((base + i) * CW, CW)
            pltpu.async_copy(obuf.at[pl.ds(p * OW, OW)],
                             out.at[pl.ds(c0 * 16, OW)], sem_out)
            return carry

        lax.fori_loop(0, nb, blk, 0)
        for _i in range(4):
            wait_out()

        @pl.when(wid == 31)
        def _():
            pltpu.sync_copy(tail_in, obuf.at[pl.ds(0, _TAIL * 16)])
            pltpu.sync_copy(obuf.at[pl.ds(0, _TAIL * 16)],
                            out.at[pl.ds(_NB_FULL * 2048, _TAIL * 16)])

    return transpose_k


def _make_sc_gather():
    mesh = plsc.VectorSubcoreMesh(
        core_axis_name="c", subcore_axis_name="s",
        num_cores=_NC, num_subcores=_NS)

    @functools.partial(
        pl.kernel,
        out_type=(
            jax.ShapeDtypeStruct((_BF, D), jnp.float32),
            jax.ShapeDtypeStruct((_BF,), jnp.float32),
        ),
        mesh=mesh,
        scratch_types=(
            pltpu.VMEM((_ROWS_PER_W, _IDX_W), jnp.int32),
            pltpu.VMEM((2 * _CG * _IDX_W, D), jnp.float32),
            pltpu.VMEM((2 * _CG * _IDX_W,), jnp.float32),
            pltpu.SemaphoreType.DMA,
            pltpu.SemaphoreType.DMA,
            pltpu.SemaphoreType.DMA,
        ),
        compiler_params=pltpu.CompilerParams(use_tc_tiling_on_sc=False),
    )
    def sc_gather(idx_hbm, emb_hbm, lin_hbm, rows_out, lin_out,
                  idx_v, rows_v, lin_v, sem_e, sem_l, sem_o):
        wid = lax.axis_index("s") * _NC + lax.axis_index("c")
        row0 = wid * _ROWS_PER_W
        CHW = _CG * _IDX_W
        pltpu.sync_copy(idx_hbm.at[pl.ds(row0, _ROWS_PER_W)], idx_v)

        def issue(g, p):
            for b in range(_CG):
                r = g * _CG + b
                pltpu.async_copy(
                    emb_hbm.at[idx_v.at[r]],
                    rows_v.at[pl.ds(p * CHW + b * _IDX_W, _IDX_W)], sem_e)
                pltpu.async_copy(
                    lin_hbm.at[idx_v.at[r]],
                    lin_v.at[pl.ds(p * CHW + b * _IDX_W, _IDX_W)], sem_l)

        def wait_in(p):
            for b in range(_CG):
                pltpu.make_async_copy(
                    emb_hbm.at[idx_v.at[0]],
                    rows_v.at[pl.ds(p * CHW + b * _IDX_W, _IDX_W)],
                    sem_e).wait()
                pltpu.make_async_copy(
                    lin_hbm.at[idx_v.at[0]],
                    lin_v.at[pl.ds(p * CHW + b * _IDX_W, _IDX_W)],
                    sem_l).wait()

        def wait_out():
            pltpu.make_async_copy(
                rows_v.at[pl.ds(0, CHW)], rows_out.at[pl.ds(0, CHW)],
                sem_o).wait()
            pltpu.make_async_copy(
                lin_v.at[pl.ds(0, CHW)], lin_out.at[pl.ds(0, CHW)],
                sem_o).wait()

        issue(0, 0)

        def chunk(g, carry):
            p = g & 1
            wait_in(p)

            @pl.when(g + 1 < _NCHUNK)
            def _():
                issue(g + 1, p ^ 1)

            @pl.when(g >= 2)
            def _():
                wait_out()

            base = (row0 + g * _CG) * _IDX_W
            pltpu.async_copy(rows_v.at[pl.ds(p * CHW, CHW)],
                             rows_out.at[pl.ds(base, CHW)], sem_o)
            pltpu.async_copy(lin_v.at[pl.ds(p * CHW, CHW)],
                             lin_out.at[pl.ds(base, CHW)], sem_o)
            return carry

        lax.fori_loop(0, _NCHUNK, chunk, 0)
        wait_out()
        wait_out()

    return sc_gather


_sc_cache = {}


def _sc_gather(idx2d, emb_table, lin_w):
    if "g" not in _sc_cache:
        _sc_cache["g"] = _make_sc_gather()
    return _sc_cache["g"](idx2d, emb_table, lin_w)


def _sc_transpose(embT, tail_in):
    if "t" not in _sc_cache:
        _sc_cache["t"] = _make_sc_transpose()
    return _sc_cache["t"](embT, tail_in)

_BB = 2048


def _tc_body(emb_ref, xv_ref, linv_ref, w1_ref, b1_ref, w2_ref, b2_ref,
             w3_ref, b3_ref, lb_ref, out_ref):
    ex = emb_ref[...]
    xv = xv_ref[...]
    f_ids = lax.broadcasted_iota(jnp.int32, (F, D_IN), 0)
    j_ids = lax.broadcasted_iota(jnp.int32, (F, D_IN), 1)
    e_mat = (lax.shift_right_logical(j_ids, 4) == f_ids).astype(jnp.float32)
    xve = jnp.dot(xv, e_mat, preferred_element_type=jnp.float32)
    ex = ex * xve
    j2 = lax.broadcasted_iota(jnp.int32, (D_IN, D), 0)
    d2 = lax.broadcasted_iota(jnp.int32, (D_IN, D), 1)
    s_mat = ((j2 & (D - 1)) == d2).astype(jnp.float32)
    s = jnp.dot(ex, s_mat, preferred_element_type=jnp.float32)
    sq = jnp.dot(ex * ex, s_mat, preferred_element_type=jnp.float32)
    fm = 0.5 * jnp.sum(s * s - sq, axis=1, keepdims=True)
    linear = jnp.sum(linv_ref[...] * xv, axis=1, keepdims=True) + lb_ref[0, 0]
    h = jnp.dot(ex, w1_ref[...], preferred_element_type=jnp.float32)
    h = jnp.maximum(h + b1_ref[...], 0.0)
    h = jnp.dot(h, w2_ref[...], preferred_element_type=jnp.float32)
    h = jnp.maximum(h + b2_ref[...], 0.0)
    mlp = jnp.dot(h, w3_ref[...], preferred_element_type=jnp.float32)
    out_ref[...] = linear + fm + mlp + b3_ref[0, 0]


def _tc_call(emb2d, xv, linv2d, w1, b1, w2, b2, w3, b3, lb):
    return pl.pallas_call(
        _tc_body,
        grid=(B // _BB,),
        in_specs=[
            pl.BlockSpec((_BB, D_IN), lambda i: (i, 0)),
            pl.BlockSpec((_BB, F), lambda i: (i, 0)),
            pl.BlockSpec((_BB, F), lambda i: (i, 0)),
            pl.BlockSpec((D_IN, 256), lambda i: (0, 0)),
            pl.BlockSpec((1, 256), lambda i: (0, 0)),
            pl.BlockSpec((256, 128), lambda i: (0, 0)),
            pl.BlockSpec((1, 128), lambda i: (0, 0)),
            pl.BlockSpec((128, 1), lambda i: (0, 0)),
            pl.BlockSpec((1, 1), lambda i: (0, 0)),
            pl.BlockSpec((1, 1), lambda i: (0, 0)),
        ],
        out_specs=pl.BlockSpec((_BB, 1), lambda i: (i, 0)),
        out_shape=jax.ShapeDtypeStruct((B, 1), jnp.float32),
        compiler_params=pltpu.CompilerParams(
            dimension_semantics=("parallel",)),
    )(emb2d, xv, linv2d, w1, b1, w2, b2, w3, b3, lb)


def kernel(x_field, x, x_val, emb_table, lin_w, lin_b, W1, b1, W2, b2, W3, b3):
    idx = x + x_field * FIELD_SIZE
    idx2d = idx.reshape(_ROWS_TOTAL, _IDX_W)
    tail = lax.slice(emb_table, (_NB_FULL * 128, 0), (V, D)).reshape(_TAIL * D)
    emb_lin = _sc_transpose(emb_table.T, tail).reshape(V, D)
    rows, linv = _sc_gather(idx2d, emb_lin, lin_w[:, 0])
    emb2d = rows.reshape(B, D_IN)
    linv2d = linv.reshape(B, F)
    out = _tc_call(
        emb2d, x_val, linv2d, W1, b1.reshape(1, 256), W2, b2.reshape(1, 128),
        W3, b3.reshape(1, 1), lin_b.reshape(1, 1))
    return out[:, 0]
